# Initial kernel scaffold; baseline (speedup 1.0000x reference)
import jax
import jax.numpy as jnp
from jax.experimental import pallas as pl


def kernel(input_ids, positions, hidden_states, embed_table, norm_weight):
    raise NotImplementedError("write your pallas kernel here")


# speedup vs baseline: 1.4518x; 1.4518x over previous
import functools

import jax
import jax.numpy as jnp
from jax import lax
from jax.experimental import pallas as pl
from jax.experimental.pallas import tpu as pltpu
from jax.experimental.pallas import tpu_sc as plsc

VOCAB_SIZE = 32000
D_MODEL = 2048
N_TOKENS = 16384
EPS = 1e-05

_NC = 2
_NS = 16
_NW = _NC * _NS
_B_PER_W = N_TOKENS // _NW
_CHUNK = 16
_NCHUNK = _B_PER_W // _CHUNK


def _sc_gather_body(idx_hbm, table_hbm, out_hbm, idx_v, buf0, buf1, sem0, sem1):
    wid = lax.axis_index("s") * _NC + lax.axis_index("c")
    base = wid * _B_PER_W
    pltpu.sync_copy(idx_hbm.at[pl.ds(base, _B_PER_W)], idx_v)

    def start(c, buf, sem):
        return pltpu.async_copy(
            table_hbm.at[idx_v.at[pl.ds(c * _CHUNK, _CHUNK)]], buf, sem)

    def wait(buf, sem):
        pltpu.make_async_copy(table_hbm.at[pl.ds(0, _CHUNK)], buf, sem).wait()

    def drain(buf, c):
        pltpu.sync_copy(buf, out_hbm.at[pl.ds(base + c * _CHUNK, _CHUNK)])

    start(0, buf0, sem0)

    @pl.loop(0, _NCHUNK - 2, step=2)
    def _(c):
        wait(buf0, sem0)
        start(c + 1, buf1, sem1)
        drain(buf0, c)
        wait(buf1, sem1)
        start(c + 2, buf0, sem0)
        drain(buf1, c + 1)

    wait(buf0, sem0)
    start(_NCHUNK - 1, buf1, sem1)
    drain(buf0, _NCHUNK - 2)
    wait(buf1, sem1)
    drain(buf1, _NCHUNK - 1)


def _make_sc_gather(interpret=False):
    return functools.partial(
        pl.kernel,
        out_type=jax.ShapeDtypeStruct((N_TOKENS, D_MODEL), jnp.float32),
        mesh=plsc.VectorSubcoreMesh(
            core_axis_name="c", subcore_axis_name="s",
            num_cores=_NC, num_subcores=_NS),
        scratch_types=[
            pltpu.VMEM((_B_PER_W,), jnp.int32),
            pltpu.VMEM((_CHUNK, D_MODEL), jnp.float32),
            pltpu.VMEM((_CHUNK, D_MODEL), jnp.float32),
            pltpu.SemaphoreType.DMA,
            pltpu.SemaphoreType.DMA,
        ],
        interpret=interpret,
    )(_sc_gather_body)


_SC_GATHER_CACHE = {}


def _sc_gather(input_ids, embed_table):
    if "k" not in _SC_GATHER_CACHE:
        _SC_GATHER_CACHE["k"] = _make_sc_gather()
    return _SC_GATHER_CACHE["k"](input_ids, embed_table)


_ROW_BLK = 256


def _rms_body(x_ref, w_ref, o_ref):
    x = x_ref[...]
    var = jnp.mean(x * x, axis=-1, keepdims=True)
    o_ref[...] = x * lax.rsqrt(var + EPS) * w_ref[...]


def _rms_norm(hidden_states, norm_weight):
    return pl.pallas_call(
        _rms_body,
        grid=(N_TOKENS // _ROW_BLK,),
        in_specs=[
            pl.BlockSpec((_ROW_BLK, D_MODEL), lambda i: (i, 0)),
            pl.BlockSpec((1, D_MODEL), lambda i: (0, 0)),
        ],
        out_specs=pl.BlockSpec((_ROW_BLK, D_MODEL), lambda i: (i, 0)),
        out_shape=jax.ShapeDtypeStruct((N_TOKENS, D_MODEL), jnp.float32),
    )(hidden_states, norm_weight.reshape(1, D_MODEL))


def kernel(input_ids, positions, hidden_states, embed_table, norm_weight):
    input_embeds = _sc_gather(input_ids, embed_table)
    normed = _rms_norm(hidden_states, norm_weight)
    return (normed, hidden_states, input_embeds)
